# baseline (speedup 1.0000x reference)
import functools

import jax
import jax.numpy as jnp
from jax import lax
from jax.experimental import pallas as pl
from jax.experimental.pallas import tpu as pltpu
from jax.experimental.pallas import tpu_sc as plsc

_N = 4096
_E = 65536
_SB = 16
_SBR = 7
_SC = 64
_SCR = 32
_H = 256
_NG = 50
_NBF = 128
_MAXEL = 90

_XW = _SB * _SC
_RW = _SBR * _SCR
_TW = _NBF + _H
_RWP = 256

_F32 = jnp.float32

_BN = 512


def _node_tables_body(x_ref, an_ref, semb_ref, temb_ref, wns_ref, wnt_ref,
                      bproj_ref, ts_ref, tt_ref):
    an = an_ref[...]
    sel = lax.broadcasted_iota(jnp.int32, (_BN, _MAXEL), 1).astype(_F32)
    oh = (an == sel).astype(_F32)
    es = jnp.dot(oh, semb_ref[...], preferred_element_type=_F32)
    et = jnp.dot(oh, temb_ref[...], preferred_element_type=_F32)
    xb = x_ref[...]
    ys = jnp.dot(xb, wns_ref[...], preferred_element_type=_F32) + bproj_ref[...]
    yt = jnp.dot(xb, wnt_ref[...], preferred_element_type=_F32)
    ts_ref[:, :_NBF] = es
    ts_ref[:, _NBF:] = ys
    tt_ref[:, :_NBF] = et
    tt_ref[:, _NBF:] = yt


def _node_tables(x_flat, an_f, src_emb, tgt_emb, wns, wnt, bproj):
    full = lambda shape: pl.BlockSpec(shape, lambda i: (0, 0))
    return pl.pallas_call(
        _node_tables_body,
        grid=(_N // _BN,),
        in_specs=[
            pl.BlockSpec((_BN, _XW), lambda i: (i, 0)),
            pl.BlockSpec((_BN, 1), lambda i: (i, 0)),
            full((_MAXEL, _NBF)),
            full((_MAXEL, _NBF)),
            full((_XW, _H)),
            full((_XW, _H)),
            full((1, _H)),
        ],
        out_specs=[
            pl.BlockSpec((_BN, _TW), lambda i: (i, 0)),
            pl.BlockSpec((_BN, _TW), lambda i: (i, 0)),
        ],
        out_shape=[
            jax.ShapeDtypeStruct((_N, _TW), _F32),
            jax.ShapeDtypeStruct((_N, _TW), _F32),
        ],
    )(x_flat, an_f, src_emb, tgt_emb, wns, wnt, bproj)


_NW = 32
_EPW = _E // _NW
_CH = 128
_NCH = _EPW // _CH

@functools.cache
def _sc_mesh():
    return plsc.VectorSubcoreMesh(core_axis_name="c", subcore_axis_name="s")


_GCH = 64
_GNCH = _EPW // _GCH


@functools.cache
def _sc_gather_kernel():
    @functools.partial(
        pl.kernel,
        out_type=jax.ShapeDtypeStruct((_E, _TW), _F32),
        mesh=_sc_mesh(),
        scratch_types=[
            pltpu.VMEM((8, _GCH), jnp.int32),
            pltpu.VMEM((8, _GCH), jnp.int32),
            pltpu.VMEM((4, _GCH, _TW), _F32),
            pltpu.SemaphoreType.DMA((8,)),
            pltpu.SemaphoreType.DMA((4,)),
            pltpu.SemaphoreType.DMA((4,)),
            pltpu.SemaphoreType.DMA((4,)),
        ],
        compiler_params=pltpu.CompilerParams(needs_layout_passes=False),
    )
    def gather(ts_hbm, tt_hbm, src_hbm, dst_hbm, g_hbm,
               idx_s, idx_t, rows, sem_i, sem_s, sem_t, sem_w):
        cid = lax.axis_index("c")
        sid = lax.axis_index("s")
        base = (sid * 2 + cid) * _EPW

        def eb(i):
            return base + i * _GCH

        def issue_idx(i, sl):
            pltpu.async_copy(src_hbm.at[pl.ds(eb(i), _GCH)], idx_s.at[sl],
                             sem_i.at[sl])
            pltpu.async_copy(dst_hbm.at[pl.ds(eb(i), _GCH)], idx_t.at[sl],
                             sem_i.at[sl])

        def wait_idx(i, sl):
            pltpu.make_async_copy(src_hbm.at[pl.ds(eb(i), _GCH)], idx_s.at[sl],
                                  sem_i.at[sl]).wait()
            pltpu.make_async_copy(dst_hbm.at[pl.ds(eb(i), _GCH)], idx_t.at[sl],
                                  sem_i.at[sl]).wait()

        for j in range(4):
            issue_idx(j, j)

        @pl.loop(0, _GNCH + 2)
        def _it(i):
            @pl.when(jnp.logical_and(i >= 2, i - 2 < _GNCH))
            def _():
                j = i - 2
                sl = j & 3
                pltpu.make_async_copy(tt_hbm.at[idx_t.at[j & 7]], rows.at[sl],
                                      sem_t.at[sl]).wait()
                pltpu.async_copy(rows.at[sl], g_hbm.at[pl.ds(eb(j), _GCH)],
                                 sem_w.at[sl])

            @pl.when(jnp.logical_and(i >= 1, i - 1 < _GNCH))
            def _():
                j = i - 1
                sl = j & 3
                pltpu.make_async_copy(ts_hbm.at[idx_s.at[j & 7]], rows.at[sl],
                                      sem_s.at[sl]).wait()
                pltpu.async_copy(tt_hbm.at[idx_t.at[j & 7]], rows.at[sl],
                                 sem_t.at[sl], add=True)

            @pl.when(i < _GNCH)
            def _():
                sl = i & 3

                @pl.when(i >= 4)
                def _():
                    j = i - 4
                    pltpu.make_async_copy(rows.at[sl],
                                          g_hbm.at[pl.ds(eb(j), _GCH)],
                                          sem_w.at[sl]).wait()

                wait_idx(i, i & 7)
                pltpu.async_copy(ts_hbm.at[idx_s.at[i & 7]], rows.at[sl],
                                 sem_s.at[sl])

                @pl.when(i + 4 < _GNCH)
                def _():
                    issue_idx(i + 4, (i + 4) & 7)

        for j in range(_GNCH - 4, _GNCH):
            pltpu.make_async_copy(rows.at[j & 3], g_hbm.at[pl.ds(eb(j), _GCH)],
                                  sem_w.at[j & 3]).wait()

    return gather


def _sc_gather(ts, tt, src, dst):
    return _sc_gather_kernel()(ts, tt, src, dst)


_BE = 1024


def _edge_mlp_body(d_ref, g_ref, wd1_ref, bd1_ref, wea_ref, bea_ref,
                   wmd_ref, bmd_ref, we1_ref, be1_ref, we2_ref, be2_ref,
                   m_ref):
    act = jax.nn.silu
    step = 6.0 / (_NG - 1)
    offs = lax.broadcasted_iota(jnp.int32, (1, _NG), 1).astype(_F32) * step
    coeff = -0.5 / step**2
    d = d_ref[...]
    g = jnp.exp(coeff * (d - offs) ** 2)
    x_dist = jnp.dot(g, wd1_ref[...], preferred_element_type=_F32) + bd1_ref[...]
    xe = act(g_ref[:, :_NBF] + x_dist)
    xe = act(jnp.dot(xe, wea_ref[...], preferred_element_type=_F32) + bea_ref[...])
    xe = act(jnp.dot(xe, wmd_ref[...], preferred_element_type=_F32) + bmd_ref[...])
    m = act(g_ref[:, _NBF:]) * xe
    m = act(jnp.dot(m, we1_ref[...], preferred_element_type=_F32) + be1_ref[...])
    m = act(jnp.dot(m, we2_ref[...], preferred_element_type=_F32) + be2_ref[...])
    mpad = jnp.concatenate([m, jnp.zeros((_BE, _RWP - _RW), _F32)], axis=1)
    m_ref[...] = mpad.T


def _edge_mlp(d2, g, wd1, bd1, wea, bea, wmd, bmd, we1, be1, we2, be2):
    full = lambda shape: pl.BlockSpec(shape, lambda i: (0, 0))
    return pl.pallas_call(
        _edge_mlp_body,
        grid=(_E // _BE,),
        in_specs=[
            pl.BlockSpec((_BE, 1), lambda i: (i, 0)),
            pl.BlockSpec((_BE, _TW), lambda i: (i, 0)),
            full((_NG, _NBF)),
            full((1, _NBF)),
            full((_NBF, _NBF)),
            full((1, _NBF)),
            full((_NBF, _H)),
            full((1, _H)),
            full((_H, _H)),
            full((1, _H)),
            full((_H, _RW)),
            full((1, _RW)),
        ],
        out_specs=[pl.BlockSpec((_RWP, _BE), lambda i: (0, i))],
        out_shape=[jax.ShapeDtypeStruct((_RWP, _E), _F32)],
    )(d2, g, wd1, bd1, wea, bea, wmd, bmd, we1, be1, we2, be2)[0]


_EH = _E // 2
_SCH = 512
_SNCH = _EH // _SCH


@functools.cache
def _sc_scatter_kernel():
    @functools.partial(
        pl.kernel,
        out_type=jax.ShapeDtypeStruct((2, _RWP, _N), _F32),
        mesh=_sc_mesh(),
        scratch_types=[
            pltpu.VMEM((2, _SCH), jnp.int32),
            pltpu.VMEM((2, 16, _SCH), _F32),
            pltpu.VMEM((16, _N), _F32),
            pltpu.SemaphoreType.DMA,
            pltpu.SemaphoreType.DMA,
        ],
        compiler_params=pltpu.CompilerParams(needs_layout_passes=False),
    )
    def scatter(m_hbm, dst_hbm, z_hbm, out_hbm, idx_v, mch, acc, sem0, sem1):
        cid = lax.axis_index("c")
        sid = lax.axis_index("s")
        col = sid * 16
        base = cid * _EH
        sems = (sem0, sem1)

        pltpu.sync_copy(z_hbm, acc)
        lane = lax.iota(jnp.int32, 16)

        def issue(cur, b):
            eb = base + cur * _SCH
            pltpu.async_copy(dst_hbm.at[pl.ds(eb, _SCH)], idx_v.at[b], sems[b])
            pltpu.async_copy(m_hbm.at[pl.ds(col, 16), pl.ds(eb, _SCH)],
                             mch.at[b], sems[b])

        def wait(cur, b):
            eb = base + cur * _SCH
            pltpu.make_async_copy(dst_hbm.at[pl.ds(eb, _SCH)], idx_v.at[b],
                                  sems[b]).wait()
            pltpu.make_async_copy(m_hbm.at[pl.ds(col, 16), pl.ds(eb, _SCH)],
                                  mch.at[b], sems[b]).wait()

        issue(0, 0)

        @pl.loop(0, _SNCH, step=2)
        def _pair(k):
            for b in range(2):
                cur = k + b

                @pl.when(cur + 1 < _SNCH)
                def _():
                    issue(cur + 1, 1 - b)

                wait(cur, b)

                @plsc.parallel_loop(0, _SCH // 16, unroll=8)
                def _grp(g):
                    dvec = idx_v[b, pl.ds(g * 16, 16)]
                    for c in range(16):
                        colp = (c + lane) & 15
                        vals = plsc.load_gather(mch.at[b], [colp, g * 16 + lane])
                        plsc.addupdate_scatter(acc, [colp, dvec], vals)

        pltpu.sync_copy(acc, out_hbm.at[cid, pl.ds(col, 16), :])

    return scatter


def _sc_scatter(m, dst, zacc):
    return _sc_scatter_kernel()(m, dst, zacc)


_BO = 1024


def _output_body(p0_ref, p1_ref, wout_ref, o_ref):
    acc = (p0_ref[...] + p1_ref[...]).T
    o_ref[...] = jnp.dot(acc, wout_ref[...], preferred_element_type=_F32)


def _output(p0, p1, wout):
    return pl.pallas_call(
        _output_body,
        grid=(_N // _BO,),
        in_specs=[
            pl.BlockSpec((_RWP, _BO), lambda i: (0, i)),
            pl.BlockSpec((_RWP, _BO), lambda i: (0, i)),
            pl.BlockSpec((_RWP, _XW), lambda i: (0, 0)),
        ],
        out_specs=[pl.BlockSpec((_BO, _XW), lambda i: (i, 0))],
        out_shape=[jax.ShapeDtypeStruct((_N, _XW), _F32)],
    )(p0, p1, wout)[0]


def kernel(x, atomic_numbers, edge_distance, edge_index, cutoff_index,
           W_dist1, b_dist1, src_emb, tgt_emb, W_eattr, b_eattr,
           W_mdist, b_mdist, W_proj, b_proj, W_e1, b_e1, W_e2, b_e2,
           W_down, W_up, R, R_inv):
    f32 = _F32
    a1 = jnp.einsum('rs,Cc->sCrc', R, W_down).reshape(_XW, _RW)
    wns = a1 @ W_proj[:_RW]
    wnt = a1 @ W_proj[_RW:]
    wout = jnp.einsum('sr,cC->rcsC', R_inv, W_up).reshape(_RW, _XW)
    wout = jnp.concatenate([wout, jnp.zeros((_RWP - _RW, _XW), f32)], axis=0)

    x_flat = x.reshape(_N, _XW).astype(f32)
    an_f = atomic_numbers.astype(f32).reshape(_N, 1)
    src = edge_index[0].astype(jnp.int32)
    dst = edge_index[1].astype(jnp.int32)

    ts, tt = _node_tables(x_flat, an_f, src_emb, tgt_emb, wns, wnt,
                          b_proj.reshape(1, _H))
    g = _sc_gather(ts, tt, src, dst)
    m = _edge_mlp(edge_distance.reshape(_E, 1), g,
                  W_dist1, b_dist1.reshape(1, _NBF),
                  W_eattr, b_eattr.reshape(1, _NBF),
                  W_mdist, b_mdist.reshape(1, _H),
                  W_e1, b_e1.reshape(1, _H),
                  W_e2, b_e2.reshape(1, _RW))
    zacc = jnp.zeros((16, _N), dtype=f32)
    partials = _sc_scatter(m, dst, zacc)
    out = _output(partials[0], partials[1], wout)
    return out.reshape(_N, _SB, _SC)


# speedup vs baseline: 1.4346x; 1.4346x over previous
import functools

import jax
import jax.numpy as jnp
from jax import lax
from jax.experimental import pallas as pl
from jax.experimental.pallas import tpu as pltpu
from jax.experimental.pallas import tpu_sc as plsc

_N = 4096
_E = 65536
_SB = 16
_SBR = 7
_SC = 64
_SCR = 32
_H = 256
_NG = 50
_NBF = 128
_MAXEL = 90

_XW = _SB * _SC
_RW = _SBR * _SCR
_TW = _NBF + _H
_RWP = 256

_F32 = jnp.float32

_BN = 512


def _node_tables_body(x_ref, an_ref, semb_ref, temb_ref, wns_ref, wnt_ref,
                      bproj_ref, ts_ref, tt_ref):
    an = an_ref[...]
    sel = lax.broadcasted_iota(jnp.int32, (_BN, _MAXEL), 1).astype(_F32)
    oh = (an == sel).astype(_F32)
    es = jnp.dot(oh, semb_ref[...], preferred_element_type=_F32)
    et = jnp.dot(oh, temb_ref[...], preferred_element_type=_F32)
    xb = x_ref[...]
    ys = jnp.dot(xb, wns_ref[...], preferred_element_type=_F32) + bproj_ref[...]
    yt = jnp.dot(xb, wnt_ref[...], preferred_element_type=_F32)
    ts_ref[:, :_NBF] = es
    ts_ref[:, _NBF:] = ys
    tt_ref[:, :_NBF] = et
    tt_ref[:, _NBF:] = yt


def _node_tables(x_flat, an_f, src_emb, tgt_emb, wns, wnt, bproj):
    full = lambda shape: pl.BlockSpec(shape, lambda i: (0, 0))
    return pl.pallas_call(
        _node_tables_body,
        grid=(_N // _BN,),
        in_specs=[
            pl.BlockSpec((_BN, _XW), lambda i: (i, 0)),
            pl.BlockSpec((_BN, 1), lambda i: (i, 0)),
            full((_MAXEL, _NBF)),
            full((_MAXEL, _NBF)),
            full((_XW, _H)),
            full((_XW, _H)),
            full((1, _H)),
        ],
        out_specs=[
            pl.BlockSpec((_BN, _TW), lambda i: (i, 0)),
            pl.BlockSpec((_BN, _TW), lambda i: (i, 0)),
        ],
        out_shape=[
            jax.ShapeDtypeStruct((_N, _TW), _F32),
            jax.ShapeDtypeStruct((_N, _TW), _F32),
        ],
    )(x_flat, an_f, src_emb, tgt_emb, wns, wnt, bproj)


_NW = 32
_EPW = _E // _NW
_CH = 128
_NCH = _EPW // _CH

@functools.cache
def _sc_mesh():
    return plsc.VectorSubcoreMesh(core_axis_name="c", subcore_axis_name="s")


_GCH = 64


@functools.cache
def _sc_gather_kernel(ne):
    epw = ne // _NW
    gnch = epw // _GCH

    @functools.partial(
        pl.kernel,
        out_type=jax.ShapeDtypeStruct((ne, _TW), _F32),
        mesh=_sc_mesh(),
        scratch_types=[
            pltpu.VMEM((8, _GCH), jnp.int32),
            pltpu.VMEM((8, _GCH), jnp.int32),
            pltpu.VMEM((4, _GCH, _TW), _F32),
            pltpu.SemaphoreType.DMA((8,)),
            pltpu.SemaphoreType.DMA((4,)),
            pltpu.SemaphoreType.DMA((4,)),
            pltpu.SemaphoreType.DMA((4,)),
        ],
        compiler_params=pltpu.CompilerParams(needs_layout_passes=False),
    )
    def gather(ts_hbm, tt_hbm, src_hbm, dst_hbm, g_hbm,
               idx_s, idx_t, rows, sem_i, sem_s, sem_t, sem_w):
        cid = lax.axis_index("c")
        sid = lax.axis_index("s")
        base = (sid * 2 + cid) * epw

        def eb(i):
            return base + i * _GCH

        def issue_idx(i, sl):
            pltpu.async_copy(src_hbm.at[pl.ds(eb(i), _GCH)], idx_s.at[sl],
                             sem_i.at[sl])
            pltpu.async_copy(dst_hbm.at[pl.ds(eb(i), _GCH)], idx_t.at[sl],
                             sem_i.at[sl])

        def wait_idx(i, sl):
            pltpu.make_async_copy(src_hbm.at[pl.ds(eb(i), _GCH)], idx_s.at[sl],
                                  sem_i.at[sl]).wait()
            pltpu.make_async_copy(dst_hbm.at[pl.ds(eb(i), _GCH)], idx_t.at[sl],
                                  sem_i.at[sl]).wait()

        for j in range(4):
            issue_idx(j, j)

        @pl.loop(0, gnch + 2)
        def _it(i):
            @pl.when(jnp.logical_and(i >= 2, i - 2 < gnch))
            def _():
                j = i - 2
                sl = j & 3
                pltpu.make_async_copy(tt_hbm.at[idx_t.at[j & 7]], rows.at[sl],
                                      sem_t.at[sl]).wait()
                pltpu.async_copy(rows.at[sl], g_hbm.at[pl.ds(eb(j), _GCH)],
                                 sem_w.at[sl])

            @pl.when(jnp.logical_and(i >= 1, i - 1 < gnch))
            def _():
                j = i - 1
                sl = j & 3
                pltpu.make_async_copy(ts_hbm.at[idx_s.at[j & 7]], rows.at[sl],
                                      sem_s.at[sl]).wait()
                pltpu.async_copy(tt_hbm.at[idx_t.at[j & 7]], rows.at[sl],
                                 sem_t.at[sl], add=True)

            @pl.when(i < gnch)
            def _():
                sl = i & 3

                @pl.when(i >= 4)
                def _():
                    j = i - 4
                    pltpu.make_async_copy(rows.at[sl],
                                          g_hbm.at[pl.ds(eb(j), _GCH)],
                                          sem_w.at[sl]).wait()

                wait_idx(i, i & 7)
                pltpu.async_copy(ts_hbm.at[idx_s.at[i & 7]], rows.at[sl],
                                 sem_s.at[sl])

                @pl.when(i + 4 < gnch)
                def _():
                    issue_idx(i + 4, (i + 4) & 7)

        for j in range(gnch - 4, gnch):
            pltpu.make_async_copy(rows.at[j & 3], g_hbm.at[pl.ds(eb(j), _GCH)],
                                  sem_w.at[j & 3]).wait()

    return gather


def _sc_gather(ts, tt, src, dst):
    return _sc_gather_kernel(src.shape[0])(ts, tt, src, dst)


_BE = 1024


def _edge_mlp_body(d_ref, g_ref, wd1_ref, bd1_ref, wea_ref, bea_ref,
                   wmd_ref, bmd_ref, we1_ref, be1_ref, we2_ref, be2_ref,
                   m_ref):
    act = jax.nn.silu
    step = 6.0 / (_NG - 1)
    offs = lax.broadcasted_iota(jnp.int32, (1, _NG), 1).astype(_F32) * step
    coeff = -0.5 / step**2
    d = d_ref[...]
    g = jnp.exp(coeff * (d - offs) ** 2)
    x_dist = jnp.dot(g, wd1_ref[...], preferred_element_type=_F32) + bd1_ref[...]
    xe = act(g_ref[:, :_NBF] + x_dist)
    xe = act(jnp.dot(xe, wea_ref[...], preferred_element_type=_F32) + bea_ref[...])
    xe = act(jnp.dot(xe, wmd_ref[...], preferred_element_type=_F32) + bmd_ref[...])
    m = act(g_ref[:, _NBF:]) * xe
    m = act(jnp.dot(m, we1_ref[...], preferred_element_type=_F32) + be1_ref[...])
    m = act(jnp.dot(m, we2_ref[...], preferred_element_type=_F32) + be2_ref[...])
    mpad = jnp.concatenate([m, jnp.zeros((_BE, _RWP - _RW), _F32)], axis=1)
    m_ref[...] = mpad.T


def _edge_mlp(d2, g, wd1, bd1, wea, bea, wmd, bmd, we1, be1, we2, be2):
    ne = d2.shape[0]
    full = lambda shape: pl.BlockSpec(shape, lambda i: (0, 0))
    return pl.pallas_call(
        _edge_mlp_body,
        grid=(ne // _BE,),
        in_specs=[
            pl.BlockSpec((_BE, 1), lambda i: (i, 0)),
            pl.BlockSpec((_BE, _TW), lambda i: (i, 0)),
            full((_NG, _NBF)),
            full((1, _NBF)),
            full((_NBF, _NBF)),
            full((1, _NBF)),
            full((_NBF, _H)),
            full((1, _H)),
            full((_H, _H)),
            full((1, _H)),
            full((_H, _RW)),
            full((1, _RW)),
        ],
        out_specs=[pl.BlockSpec((_RWP, _BE), lambda i: (0, i))],
        out_shape=[jax.ShapeDtypeStruct((_RWP, ne), _F32)],
    )(d2, g, wd1, bd1, wea, bea, wmd, bmd, we1, be1, we2, be2)[0]


_SCH = 512


@functools.cache
def _sc_scatter_kernel(ne):
    eh = ne // 2
    snch = eh // _SCH

    @functools.partial(
        pl.kernel,
        out_type=jax.ShapeDtypeStruct((2, _RWP, _N), _F32),
        mesh=_sc_mesh(),
        scratch_types=[
            pltpu.VMEM((2, _SCH), jnp.int32),
            pltpu.VMEM((2, 16, _SCH), _F32),
            pltpu.VMEM((16, _N), _F32),
            pltpu.SemaphoreType.DMA,
            pltpu.SemaphoreType.DMA,
        ],
        compiler_params=pltpu.CompilerParams(needs_layout_passes=False),
    )
    def scatter(m_hbm, dst_hbm, z_hbm, out_hbm, idx_v, mch, acc, sem0, sem1):
        cid = lax.axis_index("c")
        sid = lax.axis_index("s")
        col = sid * 16
        base = cid * eh
        sems = (sem0, sem1)

        pltpu.sync_copy(z_hbm, acc)
        lane = lax.iota(jnp.int32, 16)

        def issue(cur, b):
            eb = base + cur * _SCH
            pltpu.async_copy(dst_hbm.at[pl.ds(eb, _SCH)], idx_v.at[b], sems[b])
            pltpu.async_copy(m_hbm.at[pl.ds(col, 16), pl.ds(eb, _SCH)],
                             mch.at[b], sems[b])

        def wait(cur, b):
            eb = base + cur * _SCH
            pltpu.make_async_copy(dst_hbm.at[pl.ds(eb, _SCH)], idx_v.at[b],
                                  sems[b]).wait()
            pltpu.make_async_copy(m_hbm.at[pl.ds(col, 16), pl.ds(eb, _SCH)],
                                  mch.at[b], sems[b]).wait()

        issue(0, 0)

        @pl.loop(0, snch, step=2)
        def _pair(k):
            for b in range(2):
                cur = k + b

                @pl.when(cur + 1 < snch)
                def _():
                    issue(cur + 1, 1 - b)

                wait(cur, b)

                @plsc.parallel_loop(0, _SCH // 16, unroll=4)
                def _grp(g):
                    dvec = idx_v[b, pl.ds(g * 16, 16)]
                    for c in range(16):
                        colp = (c + lane) & 15
                        vals = plsc.load_gather(mch.at[b], [colp, g * 16 + lane])
                        plsc.addupdate_scatter(acc, [colp, dvec], vals)

        pltpu.sync_copy(acc, out_hbm.at[cid, pl.ds(col, 16), :])

    return scatter


def _sc_scatter(m, dst, zacc):
    return _sc_scatter_kernel(dst.shape[0])(m, dst, zacc)


_BO = 1024


def _output_body(p0_ref, p1_ref, p2_ref, p3_ref, wout_ref, o_ref):
    acc = (p0_ref[...] + p1_ref[...] + p2_ref[...] + p3_ref[...]).T
    o_ref[...] = jnp.dot(acc, wout_ref[...], preferred_element_type=_F32)


def _output(ps, wout):
    pspec = pl.BlockSpec((_RWP, _BO), lambda i: (0, i))
    return pl.pallas_call(
        _output_body,
        grid=(_N // _BO,),
        in_specs=[pspec, pspec, pspec, pspec,
                  pl.BlockSpec((_RWP, _XW), lambda i: (0, 0))],
        out_specs=[pl.BlockSpec((_BO, _XW), lambda i: (i, 0))],
        out_shape=[jax.ShapeDtypeStruct((_N, _XW), _F32)],
    )(*ps, wout)[0]


def kernel(x, atomic_numbers, edge_distance, edge_index, cutoff_index,
           W_dist1, b_dist1, src_emb, tgt_emb, W_eattr, b_eattr,
           W_mdist, b_mdist, W_proj, b_proj, W_e1, b_e1, W_e2, b_e2,
           W_down, W_up, R, R_inv):
    f32 = _F32
    a1 = jnp.einsum('rs,Cc->sCrc', R, W_down).reshape(_XW, _RW)
    wns = a1 @ W_proj[:_RW]
    wnt = a1 @ W_proj[_RW:]
    wout = jnp.einsum('sr,cC->rcsC', R_inv, W_up).reshape(_RW, _XW)
    wout = jnp.concatenate([wout, jnp.zeros((_RWP - _RW, _XW), f32)], axis=0)

    x_flat = x.reshape(_N, _XW).astype(f32)
    an_f = atomic_numbers.astype(f32).reshape(_N, 1)
    src = edge_index[0].astype(jnp.int32)
    dst = edge_index[1].astype(jnp.int32)

    ts, tt = _node_tables(x_flat, an_f, src_emb, tgt_emb, wns, wnt,
                          b_proj.reshape(1, _H))
    zacc = jnp.zeros((16, _N), dtype=f32)
    d2 = edge_distance.reshape(_E, 1)

    half = _E // 2
    ps = []
    for h in range(2):
        sl = slice(h * half, (h + 1) * half)
        g = _sc_gather(ts, tt, src[sl], dst[sl])
        m = _edge_mlp(d2[sl], g,
                      W_dist1, b_dist1.reshape(1, _NBF),
                      W_eattr, b_eattr.reshape(1, _NBF),
                      W_mdist, b_mdist.reshape(1, _H),
                      W_e1, b_e1.reshape(1, _H),
                      W_e2, b_e2.reshape(1, _RW))
        partials = _sc_scatter(m, dst[sl], zacc)
        ps.extend([partials[0], partials[1]])
    out = _output(ps, wout)
    return out.reshape(_N, _SB, _SC)
